# Initial kernel scaffold; baseline (speedup 1.0000x reference)
import jax
import jax.numpy as jnp
from jax.experimental import pallas as pl


def kernel(x, addresses, contents, Wa, Wr):
    raise NotImplementedError("write your pallas kernel here")


# speedup vs baseline: 23.1976x; 23.1976x over previous
import functools

import jax
import jax.numpy as jnp
from jax import lax
from jax.experimental import pallas as pl
from jax.experimental.pallas import tpu as pltpu
from jax.experimental.pallas import tpu_sc as plsc

HIDDEN = 1024
SLOTS = 65536
N = 64
TOPK = 4
ETA = 0.1
EPS = 1e-8

BS = 2048
NB = SLOTS // BS
KPAD = 8
NEG = -1e30


def _topk_body(x_ref, wa_ref, addr_ref, outv_ref, outi_ref, qn_ref, rv_ref, ri_ref):
    j = pl.program_id(0)

    @pl.when(j == 0)
    def _init():
        q = lax.dot_general(x_ref[...], wa_ref[...], (((1,), (1,)), ((), ())),
                            preferred_element_type=jnp.float32)
        qnorm = jnp.sqrt(jnp.sum(q * q, axis=1, keepdims=True))
        qn_ref[...] = q / jnp.clip(qnorm, EPS, None)
        rv_ref[...] = jnp.full((KPAD, N), NEG, jnp.float32)
        ri_ref[...] = jnp.zeros((KPAD, N), jnp.int32)

    a = addr_ref[...]
    anorm = jnp.sqrt(jnp.sum(a * a, axis=1, keepdims=True))
    an = a / jnp.clip(anorm, EPS, None)
    sT = lax.dot_general(an, qn_ref[...], (((1,), (1,)), ((), ())),
                         preferred_element_type=jnp.float32)

    iota_s = lax.broadcasted_iota(jnp.int32, (BS, N), 0)
    bv = []
    bi = []
    for _ in range(TOPK):
        m = jnp.max(sT, axis=0)
        hit = sT == m[None, :]
        am = jnp.min(jnp.where(hit, iota_s, SLOTS), axis=0)
        bv.append(m)
        bi.append(am + j * BS)
        sT = jnp.where(iota_s == am[None, :], NEG, sT)

    cat_v = jnp.concatenate(
        [rv_ref[0:TOPK, :]] + [v[None, :] for v in bv], axis=0)
    cat_i = jnp.concatenate(
        [ri_ref[0:TOPK, :]] + [i[None, :] for i in bi], axis=0)
    iota_c = lax.broadcasted_iota(jnp.int32, (2 * TOPK, N), 0)
    nv = []
    ni = []
    for _ in range(TOPK):
        m = jnp.max(cat_v, axis=0)
        hit = cat_v == m[None, :]
        pos = jnp.min(jnp.where(hit, iota_c, 2 * TOPK), axis=0)
        sel = iota_c == pos[None, :]
        nv.append(m)
        ni.append(jnp.sum(jnp.where(sel, cat_i, 0), axis=0))
        cat_v = jnp.where(sel, NEG, cat_v)
    pad_v = [jnp.full((N,), NEG, jnp.float32)[None, :]] * (KPAD - TOPK)
    pad_i = [jnp.zeros((N,), jnp.int32)[None, :]] * (KPAD - TOPK)
    rv_ref[...] = jnp.concatenate([v[None, :] for v in nv] + pad_v, axis=0)
    ri_ref[...] = jnp.concatenate([i[None, :] for i in ni] + pad_i, axis=0)

    @pl.when(j == NB - 1)
    def _out():
        outv_ref[...] = rv_ref[...]
        outi_ref[...] = ri_ref[...]


def _topk_call(x, addresses, Wa):
    return pl.pallas_call(
        _topk_body,
        grid=(NB,),
        in_specs=[
            pl.BlockSpec((N, HIDDEN), lambda j: (0, 0)),
            pl.BlockSpec((HIDDEN, HIDDEN), lambda j: (0, 0)),
            pl.BlockSpec((BS, HIDDEN), lambda j: (j, 0)),
        ],
        out_specs=[
            pl.BlockSpec((KPAD, N), lambda j: (0, 0)),
            pl.BlockSpec((KPAD, N), lambda j: (0, 0)),
        ],
        out_shape=[
            jax.ShapeDtypeStruct((KPAD, N), jnp.float32),
            jax.ShapeDtypeStruct((KPAD, N), jnp.int32),
        ],
        scratch_shapes=[
            pltpu.VMEM((N, HIDDEN), jnp.float32),
            pltpu.VMEM((KPAD, N), jnp.float32),
            pltpu.VMEM((KPAD, N), jnp.int32),
        ],
    )(x, Wa, addresses)


_NW = 32
_BPW = (N * TOPK) // _NW


def _gather_sc(contents, idx):
    mesh = plsc.VectorSubcoreMesh(core_axis_name="c", subcore_axis_name="s")

    @functools.partial(
        pl.kernel, mesh=mesh,
        out_type=jax.ShapeDtypeStruct((N * TOPK, HIDDEN), jnp.float32),
        scratch_types=[
            pltpu.VMEM((_BPW,), jnp.int32),
            pltpu.VMEM((_BPW, HIDDEN), jnp.float32),
            pltpu.SemaphoreType.DMA,
        ],
    )
    def gk(table_hbm, idx_hbm, out_hbm, idx_v, rows_v, sem):
        wid = lax.axis_index("s") * 2 + lax.axis_index("c")
        base = wid * _BPW
        pltpu.sync_copy(idx_hbm.at[pl.ds(base, _BPW)], idx_v)
        pltpu.async_copy(table_hbm.at[idx_v], rows_v, sem).wait()
        pltpu.sync_copy(rows_v, out_hbm.at[pl.ds(base, _BPW)])

    return gk(contents, idx)


def _combine_body(x_ref, ti_ref, tiT_ref, tvT_ref, g_ref, wr_ref, out_ref):
    x = x_ref[...]
    best_row = ti_ref[0:1, :]
    tvT = tvT_ref[...]
    m = jnp.max(tvT, axis=1, keepdims=True)
    e = jnp.exp(tvT - m)
    w = e / jnp.sum(e, axis=1, keepdims=True)

    read = jnp.zeros((N, HIDDEN), jnp.float32)
    for k in range(TOPK):
        ti_col = tiT_ref[:, k:k + 1]
        match = (ti_col == best_row).astype(jnp.float32)
        counts = jnp.sum(match, axis=1, keepdims=True)
        sums = lax.dot_general(match, x, (((1,), (0,)), ((), ())),
                               preferred_element_type=jnp.float32)
        mask = (counts > 0).astype(jnp.float32)
        mean_w = sums / jnp.clip(counts, 1.0, None)
        g_k = g_ref[k * N:(k + 1) * N, :]
        upd = g_k * (1.0 - ETA * mask) + ETA * mask * mean_w
        read = read + w[:, k:k + 1] * upd
    out_ref[...] = lax.dot_general(read, wr_ref[...], (((1,), (1,)), ((), ())),
                                   preferred_element_type=jnp.float32)


def _combine_call(x, topi, topiT, topvT, gathered, Wr):
    return pl.pallas_call(
        _combine_body,
        out_shape=jax.ShapeDtypeStruct((N, HIDDEN), jnp.float32),
    )(x, topi, topiT, topvT, gathered, Wr)


def kernel(x, addresses, contents, Wa, Wr):
    topv, topi = _topk_call(x, addresses, Wa)
    idx_flat = topi[0:TOPK, :].reshape(-1)
    gathered = _gather_sc(contents, idx_flat)
    topiT = topi[0:TOPK, :].T
    topvT = topv[0:TOPK, :].T
    return _combine_call(x, topi, topiT, topvT, gathered, Wr)
